# Initial kernel scaffold; baseline (speedup 1.0000x reference)
import jax
import jax.numpy as jnp
from jax.experimental import pallas as pl


def kernel(user_text, item_text, user_image, item_image, W1, b1, W2, b2, Wi1, bi1, Wi2, bi2, emb_user, emb_item, edge_weight, edge_index, users):
    raise NotImplementedError("write your pallas kernel here")


# speedup vs baseline: 5.1962x; 5.1962x over previous
import functools

import jax
import jax.numpy as jnp
from jax import lax
from jax.experimental import pallas as pl
from jax.experimental.pallas import tpu as pltpu
from jax.experimental.pallas import tpu_sc as plsc

NUM_USERS = 4000
NUM_ITEMS = 6000
NN = 10000
NPAD = 10240
D = 128
NE = 640000
NW = 32
EB = 128
CPB = 16
NCH = 10
EPW = NCH * CPB * EB
YROWS = 640
HID_T = 614
HID_TP = 640

_mesh = plsc.VectorSubcoreMesh(core_axis_name="c", subcore_axis_name="s")


def _mlp_body(t_ref, im_ref, w1, b1, w2, b2, wi1, bi1, wi2, bi2, ids_ref, o_ref):
    h = jnp.dot(t_ref[...], w1[...], preferred_element_type=jnp.float32) + b1[...]
    h = jnp.where(h >= 0, h, 0.01 * h)
    te = jnp.dot(h, w2[...], preferred_element_type=jnp.float32) + b2[...]
    hi = jnp.dot(im_ref[...], wi1[...], preferred_element_type=jnp.float32) + bi1[...]
    hi = jnp.where(hi >= 0, hi, 0.01 * hi)
    ie = jnp.dot(hi, wi2[...], preferred_element_type=jnp.float32) + bi2[...]
    tn = te / jnp.maximum(jnp.sqrt(jnp.sum(te * te, axis=1, keepdims=True)), 1e-12)
    im = ie / jnp.maximum(jnp.sqrt(jnp.sum(ie * ie, axis=1, keepdims=True)), 1e-12)
    o_ref[...] = jnp.concatenate([tn + im, ids_ref[...]], axis=1)


def _mlp_call(T, IM, W1p, b1p, W2p, b2, Wi1, bi1, Wi2, bi2, ids):
    rb = 400
    return pl.pallas_call(
        _mlp_body,
        grid=(NN // rb,),
        in_specs=[
            pl.BlockSpec((rb, 2048), lambda i: (i, 0)),
            pl.BlockSpec((rb, 512), lambda i: (i, 0)),
            pl.BlockSpec((2048, HID_TP), lambda i: (0, 0)),
            pl.BlockSpec((1, HID_TP), lambda i: (0, 0)),
            pl.BlockSpec((HID_TP, 64), lambda i: (0, 0)),
            pl.BlockSpec((1, 64), lambda i: (0, 0)),
            pl.BlockSpec((512, 128), lambda i: (0, 0)),
            pl.BlockSpec((1, 128), lambda i: (0, 0)),
            pl.BlockSpec((128, 64), lambda i: (0, 0)),
            pl.BlockSpec((1, 64), lambda i: (0, 0)),
            pl.BlockSpec((rb, 64), lambda i: (i, 0)),
        ],
        out_specs=pl.BlockSpec((rb, D), lambda i: (i, 0)),
        out_shape=jax.ShapeDtypeStruct((NN, D), jnp.float32),
    )(T, IM, W1p, b1p, W2p, b2, Wi1, bi1, Wi2, bi2, ids)


@functools.partial(
    pl.kernel,
    out_type=(
        jax.ShapeDtypeStruct((NPAD, D), jnp.float32),
        jax.ShapeDtypeStruct((NPAD, D), jnp.float32),
    ),
    mesh=_mesh,
    scratch_types=[
        pltpu.VMEM((CPB, EB), jnp.int32),
        pltpu.VMEM((CPB, EB), jnp.int32),
        pltpu.VMEM((CPB, EB), jnp.float32),
        pltpu.VMEM((EB, D), jnp.float32),
        pltpu.VMEM_SHARED((NPAD, D), jnp.float32),
        pltpu.SemaphoreType.DMA,
    ],
)
def _scatter_k(x_hbm, src_hbm, dst_hbm, w_hbm, z_hbm,
               p0_hbm, p1_hbm,
               src_v, dst_v, w_v, rows_v, y_sh, sem):
    c = lax.axis_index("c")
    s = lax.axis_index("s")
    wid = s * 2 + c
    pltpu.sync_copy(z_hbm, y_sh.at[pl.ds(s * YROWS, YROWS)])
    plsc.subcore_barrier()

    def chunk(ch, carry0):
        pltpu.sync_copy(src_hbm.at[wid, ch], src_v)
        pltpu.sync_copy(dst_hbm.at[wid, ch], dst_v)
        pltpu.sync_copy(w_hbm.at[wid, ch], w_v)

        def blk(j, carry):
            pltpu.async_copy(x_hbm.at[src_v.at[j]], rows_v, sem).wait()

            def grp16(g, carry2):
                wv = w_v[j, pl.ds(g * 16, 16)]
                for l in range(16):
                    e = g * 16 + l
                    wsc = wv[l]
                    for d in range(8):
                        sl = pl.ds(d * 16, 16)
                        rows_v[e, sl] = rows_v[e, sl] * wsc
                return carry2

            lax.fori_loop(0, EB // 16, grp16, 0)
            pltpu.sync_copy(rows_v, y_sh.at[dst_v.at[j]], add=True)
            return carry

        lax.fori_loop(0, CPB, blk, 0)
        return carry0

    lax.fori_loop(0, NCH, chunk, 0)
    plsc.subcore_barrier()
    rs = pl.ds(s * YROWS, YROWS)

    @pl.when(c == 0)
    def _():
        pltpu.sync_copy(y_sh.at[rs], p0_hbm.at[rs])

    @pl.when(c == 1)
    def _():
        pltpu.sync_copy(y_sh.at[rs], p1_hbm.at[rs])


@functools.partial(
    pl.kernel,
    out_type=(
        jax.ShapeDtypeStruct((NPAD, D), jnp.float32),
        jax.ShapeDtypeStruct((NPAD, D), jnp.float32),
        jax.ShapeDtypeStruct((512, D), jnp.float32),
    ),
    mesh=_mesh,
    scratch_types=[
        pltpu.VMEM((80, D), jnp.float32),
        pltpu.VMEM((80, D), jnp.float32),
        pltpu.VMEM((80, D), jnp.float32),
        pltpu.VMEM((16,), jnp.int32),
        pltpu.VMEM((16, D), jnp.float32),
        pltpu.VMEM((16, D), jnp.float32),
        pltpu.SemaphoreType.DMA,
    ],
)
def _combine_k(a_hbm, q0_hbm, q1_hbm, users_hbm,
               xn_hbm, acc_hbm, us_hbm,
               a_v, p0_v, p1_v, uidx_v, urow_v, uacc_v, sem):
    c = lax.axis_index("c")
    s = lax.axis_index("s")
    wid = s * 2 + c
    base = wid * (NPAD // NW)

    def chunk(k, carry):
        off = base + k * 80
        rs = pl.ds(off, 80)
        pltpu.sync_copy(a_hbm.at[rs], a_v)
        pltpu.sync_copy(q0_hbm.at[rs], p0_v)
        pltpu.sync_copy(q1_hbm.at[rs], p1_v)

        def row(r, carry2):
            for d in range(8):
                sl = pl.ds(d * 16, 16)
                xv = p0_v[r, sl] + p1_v[r, sl]
                p0_v[r, sl] = xv
                a_v[r, sl] = a_v[r, sl] + xv
            return carry2

        lax.fori_loop(0, 80, row, 0)
        pltpu.sync_copy(p0_v, xn_hbm.at[rs])
        pltpu.sync_copy(a_v, acc_hbm.at[rs])
        return carry

    lax.fori_loop(0, (NPAD // NW) // 80, chunk, 0)

    ub = wid * 16
    pltpu.sync_copy(users_hbm.at[pl.ds(ub, 16)], uidx_v)
    pltpu.async_copy(a_hbm.at[uidx_v], uacc_v, sem).wait()
    for q_hbm in (q0_hbm, q1_hbm):
        pltpu.async_copy(q_hbm.at[uidx_v], urow_v, sem).wait()

        def urow_add(r, carry):
            for d in range(8):
                sl = pl.ds(d * 16, 16)
                uacc_v[r, sl] = uacc_v[r, sl] + urow_v[r, sl]
            return carry

        lax.fori_loop(0, 16, urow_add, 0)
    pltpu.sync_copy(uacc_v, us_hbm.at[pl.ds(ub, 16)])


def _rate_body(u_ref, it_ref, o_ref):
    logits = lax.dot_general(
        u_ref[...], it_ref[...], (((1,), (1,)), ((), ()))) * (1.0 / 16.0)
    o_ref[...] = 1.0 / (1.0 + jnp.exp(-logits))


def _rate_call(users_sel, items):
    ib = 128
    ni = items.shape[0]
    return pl.pallas_call(
        _rate_body,
        grid=(ni // ib,),
        in_specs=[
            pl.BlockSpec((512, D), lambda i: (0, 0)),
            pl.BlockSpec((ib, D), lambda i: (i, 0)),
        ],
        out_specs=pl.BlockSpec((512, ib), lambda i: (0, i)),
        out_shape=jax.ShapeDtypeStruct((512, ni), jnp.float32),
    )(users_sel, items)


def kernel(user_text, item_text, user_image, item_image, W1, b1, W2, b2,
           Wi1, bi1, Wi2, bi2, emb_user, emb_item, edge_weight, edge_index,
           users):
    T = jnp.concatenate([user_text, item_text], axis=0)
    IM = jnp.concatenate([user_image, item_image], axis=0)
    ids = jnp.concatenate([emb_user, emb_item], axis=0)
    W1p = jnp.pad(W1, ((0, 0), (0, HID_TP - HID_T)))
    b1p = jnp.pad(b1, (0, HID_TP - HID_T)).reshape(1, HID_TP)
    W2p = jnp.pad(W2, ((0, HID_TP - HID_T), (0, 0)))

    X0 = _mlp_call(T, IM, W1p, b1p, W2p, b2.reshape(1, 64), Wi1,
                   bi1.reshape(1, 128), Wi2, bi2.reshape(1, 64), ids)
    X0p = jnp.pad(X0, ((0, NPAD - NN), (0, 0)))

    pad = NW * EPW - NE
    src3 = jnp.pad(edge_index[0], (0, pad)).reshape(NW, NCH, CPB, EB)
    dst3 = jnp.pad(edge_index[1], (0, pad)).reshape(NW, NCH, CPB, EB)
    w3 = jnp.pad(edge_weight, (0, pad)).reshape(NW, NCH, CPB, EB)
    Z = jnp.zeros((YROWS, D), jnp.float32)

    acc = X0p
    x_cur = X0p
    users_sel = None
    for _ in range(3):
        p0, p1 = _scatter_k(x_cur, src3, dst3, w3, Z)
        x_cur, acc, users_sel = _combine_k(acc, p0, p1, users)

    items = lax.slice(acc, (NUM_USERS, 0), (NUM_USERS + 6016, D))
    rating = _rate_call(users_sel, items)
    return rating[:, :6000]
